# baseline (speedup 1.0000x reference)
import jax
import jax.numpy as jnp
from jax import lax
from jax.experimental import pallas as pl
from jax.experimental.pallas import tpu as pltpu
from jax.experimental.pallas import tpu_sc as plsc

VOCAB = 1000000
EMB = 64
SEQ = 200
BATCH = 1024

NC = 2
NS = 16
L = 16
NW = NC * NS

V = 256
PR = V // 2
MAIN_WORDS = 999424
SLABS = MAIN_WORDS // V
SLABS_PW = SLABS // NW
TAIL_ROWS = (VOCAB - MAIN_WORDS) // 2
TAIL0 = MAIN_WORDS // 2

BB = BATCH // 8
NTB = SEQ // 8


def _iota16(off):
    return jnp.arange(16, dtype=jnp.int32) + off


def _p1_body(wtT, tailP, wt2, slab0, slab1, ov0, ov1, is0, is1, os0, os1):
    wid = lax.axis_index("s") * NC + lax.axis_index("c")
    base_s = wid * SLABS_PW
    slabs = [slab0, slab1]
    outs = [ov0, ov1]
    isems = [is0, is1]
    osems = [os0, os1]
    evs = [_iota16(16 * j) for j in range(4)]

    def in_copy(k, p):
        v0 = pl.multiple_of((base_s + k) * V, V)
        return pltpu.make_async_copy(wtT.at[:, pl.ds(v0, V)], slabs[p], isems[p])

    def out_copy(k, p):
        r0 = pl.multiple_of((base_s + k) * PR, PR)
        return pltpu.make_async_copy(outs[p], wt2.at[pl.ds(r0, PR)], osems[p])

    def transpose_slab(slab, out):
        def rbody(r, c):
            d0 = 2 * r
            for j in range(8):
                dv = jnp.full((16,), d0 + (1 if j >= 4 else 0), jnp.int32)
                g = plsc.load_gather(slab, [evs[j % 4], dv])
                out[r, pl.ds(16 * j, 16)] = g
            return c

        lax.fori_loop(0, PR, rbody, 0)

    in_copy(0, 0).start()
    in_copy(1, 1).start()

    def m_body(m, carry):
        for p in (0, 1):
            k = 2 * m + p
            in_copy(k, p).wait()

            @pl.when(m > 0)
            def _():
                out_copy(k - 2, p).wait()

            transpose_slab(slabs[p], outs[p])
            out_copy(k, p).start()

            @pl.when(k + 2 < SLABS_PW)
            def _():
                in_copy(k + 2, p).start()
        return carry

    lax.fori_loop(0, SLABS_PW // 2, m_body, 0)
    out_copy(SLABS_PW - 2, 0).wait()
    out_copy(SLABS_PW - 1, 1).wait()

    @pl.when(wid == 0)
    def _():
        for kk in range(3):
            pltpu.sync_copy(tailP.at[pl.ds(96 * kk, 96)], ov0.at[pl.ds(0, 96)])
            pltpu.sync_copy(
                ov0.at[pl.ds(0, 96)], wt2.at[pl.ds(TAIL0 + 96 * kk, 96)]
            )


def _p2_body(sent, wt2, pos128, outT, posv, idxv, ip0, ip1, rows0, rows1, ov,
             gs0, gs1):
    wid = lax.axis_index("s") * NC + lax.axis_index("c")
    tg = wid // 8
    b0 = pl.multiple_of((wid % 8) * BB, BB)
    pltpu.sync_copy(pos128, posv)
    bvs = [_iota16(16 * j) for j in range(8)]
    ips = [ip0, ip1]
    rows = [rows0, rows1]
    gsems = [gs0, gs1]

    def gather(p):
        return pltpu.make_async_copy(wt2.at[ips[p]], rows[p], gsems[p])

    def prep(dt, p):
        pars = []
        for jb in range(8):
            v = idxv[dt, pl.ds(16 * jb, 16)]
            ips[p][pl.ds(16 * jb, 16)] = v >> 1
            pars.append((v & 1) << 6)
        return pars

    def tadd(p, pars, t):
        def ebody(e, c):
            pe = plsc.load_gather(
                posv, [jnp.full((16,), t, jnp.int32), jnp.full((16,), e, jnp.int32)]
            )
            for jb in range(8):
                g = plsc.load_gather(rows[p], [bvs[jb], pars[jb] + e])
                ov[e, pl.ds(16 * jb, 16)] = g + pe
            return c

        lax.fori_loop(0, EMB, ebody, 0)

    nb = jnp.where(tg == 0, (NTB + 3) // 4, NTB // 4)

    def block(i, carry):
        k = tg + 4 * i
        t0 = pl.multiple_of(8 * k, 8)
        pltpu.sync_copy(sent.at[pl.ds(t0, 8), pl.ds(b0, BB)], idxv)
        prev = None
        for dt in range(8):
            p = dt % 2
            pars = prep(dt, p)
            gather(p).start()
            if prev is not None:
                q, qpars = prev
                gather(q).wait()
                tadd(q, qpars, t0 + dt - 1)
                pltpu.sync_copy(ov, outT.at[t0 + dt - 1, :, pl.ds(b0, BB)])
            prev = (p, pars)
        q, qpars = prev
        gather(q).wait()
        tadd(q, qpars, t0 + 7)
        pltpu.sync_copy(ov, outT.at[t0 + 7, :, pl.ds(b0, BB)])
        return carry

    lax.fori_loop(0, nb, block, 0)


def kernel(sentence, word_table, pos_table):
    wtT = jnp.transpose(word_table, (1, 0))
    tailP = word_table[MAIN_WORDS:].reshape(TAIL_ROWS, 128)
    pos128 = jnp.pad(
        lax.slice_in_dim(pos_table, 1, SEQ + 1, axis=0), ((0, 0), (0, 128 - EMB))
    )
    mesh = plsc.VectorSubcoreMesh(core_axis_name="c", subcore_axis_name="s")
    cp = pltpu.CompilerParams(use_tc_tiling_on_sc=True, needs_layout_passes=False)

    wt2 = pl.kernel(
        _p1_body,
        out_type=jax.ShapeDtypeStruct((VOCAB // 2, 128), jnp.float32),
        mesh=mesh,
        compiler_params=cp,
        scratch_types=[
            pltpu.VMEM((EMB, V), jnp.float32),
            pltpu.VMEM((EMB, V), jnp.float32),
            pltpu.VMEM((PR, 128), jnp.float32),
            pltpu.VMEM((PR, 128), jnp.float32),
            pltpu.SemaphoreType.DMA,
            pltpu.SemaphoreType.DMA,
            pltpu.SemaphoreType.DMA,
            pltpu.SemaphoreType.DMA,
        ],
    )(wtT, tailP)

    outT = pl.kernel(
        _p2_body,
        out_type=jax.ShapeDtypeStruct((SEQ, EMB, BATCH), jnp.float32),
        mesh=mesh,
        compiler_params=cp,
        scratch_types=[
            pltpu.VMEM((SEQ, 128), jnp.float32),
            pltpu.VMEM((8, BB), jnp.int32),
            pltpu.VMEM((BB,), jnp.int32),
            pltpu.VMEM((BB,), jnp.int32),
            pltpu.VMEM((BB, 128), jnp.float32),
            pltpu.VMEM((BB, 128), jnp.float32),
            pltpu.VMEM((EMB, BB), jnp.float32),
            pltpu.SemaphoreType.DMA,
            pltpu.SemaphoreType.DMA,
        ],
    )(sentence, wt2, pos128)

    return jnp.transpose(outT, (2, 0, 1))


# speedup vs baseline: 3.2392x; 3.2392x over previous
import jax
import jax.numpy as jnp
from jax import lax
from jax.experimental import pallas as pl
from jax.experimental.pallas import tpu as pltpu
from jax.experimental.pallas import tpu_sc as plsc

VOCAB = 1000000
EMB = 64
SEQ = 200
BATCH = 1024

NC = 2
NS = 16
L = 16
NW = NC * NS
ROWS = SEQ * BATCH
RPW = ROWS // NW
CHUNK = 800
NCHUNK = RPW // CHUNK
G = 100
NG = CHUNK // G
BPC = CHUNK // SEQ


def _emb_body(idx_hbm, table_hbm, pos_hbm, out_hbm, idx_v, rows_v, pos_v, sem):
    wid = lax.axis_index("s") * NC + lax.axis_index("c")
    base = wid * RPW
    pltpu.sync_copy(pos_hbm, pos_v)
    for c in range(NCHUNK):
        cbase = pl.multiple_of(base + c * CHUNK, CHUNK)
        pltpu.sync_copy(idx_hbm.at[pl.ds(pl.multiple_of(cbase // G, NG), NG)], idx_v)
        copies = [
            pltpu.async_copy(
                table_hbm.at[idx_v.at[j]], rows_v.at[pl.ds(j * G, G)], sem
            )
            for j in range(NG)
        ]
        for cp in copies:
            cp.wait()

        def body(t, carry):
            for j in range(EMB // L):
                p = pos_v[t, pl.ds(j * L, L)]
                for b in range(BPC):
                    r = b * SEQ + t
                    rows_v[r, pl.ds(j * L, L)] = rows_v[r, pl.ds(j * L, L)] + p
            return carry

        lax.fori_loop(0, SEQ, body, 0)
        pltpu.sync_copy(rows_v, out_hbm.at[pl.ds(cbase, CHUNK)])


W = 8192
NTP = (VOCAB + W - 1) // W


def _tp_body(x_ref, o_ref):
    xt = x_ref[...].T.reshape(W // 2, 2, EMB)
    o_ref[:, 0:64] = xt[:, 0, :]
    o_ref[:, 64:128] = xt[:, 1, :]


def _tc_relayout(wtT):
    return pl.pallas_call(
        _tp_body,
        grid=(NTP,),
        in_specs=[pl.BlockSpec((EMB, W), lambda i: (0, i))],
        out_specs=pl.BlockSpec((W // 2, 128), lambda i: (i, 0)),
        out_shape=jax.ShapeDtypeStruct((VOCAB // 2, 128), jnp.float32),
    )(wtT)


def kernel(sentence, word_table, pos_table):
    wt_r = _tc_relayout(jnp.transpose(word_table, (1, 0)))
    wt_lin = jnp.reshape(wt_r, (VOCAB, EMB))
    idx = jnp.transpose(sentence, (1, 0)).reshape(ROWS // G, G)
    pos = lax.slice_in_dim(pos_table, 1, SEQ + 1, axis=0)
    mesh = plsc.VectorSubcoreMesh(core_axis_name="c", subcore_axis_name="s")
    out = pl.kernel(
        _emb_body,
        out_type=jax.ShapeDtypeStruct((ROWS, EMB), jnp.float32),
        mesh=mesh,
        compiler_params=pltpu.CompilerParams(use_tc_tiling_on_sc=False),
        scratch_types=[
            pltpu.VMEM((NG, G), jnp.int32),
            pltpu.VMEM((CHUNK, EMB), jnp.float32),
            pltpu.VMEM((SEQ, EMB), jnp.float32),
            pltpu.SemaphoreType.DMA,
        ],
    )(idx, wt_lin, pos)
    return out.reshape(BATCH, SEQ, EMB)


# speedup vs baseline: 3.5869x; 1.1073x over previous
import jax
import jax.numpy as jnp
from jax import lax
from jax.experimental import pallas as pl
from jax.experimental.pallas import tpu as pltpu
from jax.experimental.pallas import tpu_sc as plsc

VOCAB = 1000000
EMB = 64
SEQ = 200
BATCH = 1024

NC = 2
NS = 16
L = 16
NW = NC * NS
ROWS = SEQ * BATCH
RPW = ROWS // NW
CHUNK = SEQ
NCHUNK = RPW // CHUNK
G = 100
NG = CHUNK // G


def _emb_body(idx_hbm, table_hbm, pos_hbm, out_hbm,
              iv0, iv1, r0v, r1v, o0v, o1v, pos_v, g0, g1, w0, w1):
    wid = lax.axis_index("s") * NC + lax.axis_index("c")
    base = wid * RPW
    ivs = [iv0, iv1]
    rows = [r0v, r1v]
    outs = [o0v, o1v]
    gsems = [g0, g1]
    osems = [w0, w1]
    pltpu.sync_copy(pos_hbm, pos_v)

    def stage_idx(grp):
        r0 = pl.multiple_of(wid * (RPW // G) + grp * 8, 8)
        pltpu.sync_copy(idx_hbm.at[pl.ds(r0, 8)], ivs[grp % 2])

    def gathers(c, p):
        return [
            pltpu.make_async_copy(
                table_hbm.at[ivs[(c // 4) % 2].at[2 * (c % 4) + k]],
                rows[p].at[pl.ds(G * k, G)],
                gsems[p],
            )
            for k in range(NG)
        ]

    def out_copy(c, p):
        cb = pl.multiple_of(base + CHUNK * c, CHUNK)
        return pltpu.make_async_copy(outs[p], out_hbm.at[pl.ds(cb, CHUNK)], osems[p])

    stage_idx(0)
    for cp in gathers(0, 0):
        cp.start()
    for c in range(NCHUNK):
        p = c % 2
        if c + 1 < NCHUNK:
            if (c + 1) % 4 == 0:
                stage_idx((c + 1) // 4)
            for cp in gathers(c + 1, (c + 1) % 2):
                cp.start()
        for cp in gathers(c, p):
            cp.wait()
        if c >= 2:
            out_copy(c - 2, p).wait()

        def body(t, carry):
            for j in range(EMB // L):
                outs[p][t, pl.ds(j * L, L)] = (
                    rows[p][t, pl.ds(j * L, L)] + pos_v[t, pl.ds(j * L, L)]
                )
            return carry

        lax.fori_loop(0, SEQ, body, 0)
        out_copy(c, p).start()
    out_copy(NCHUNK - 2, 0).wait()
    out_copy(NCHUNK - 1, 1).wait()


W = 8192
NTP = (VOCAB + W - 1) // W


def _tp_body(x_ref, o_ref):
    o_ref[:, 0:64] = x_ref[...].T


def _tc_relayout(wtT):
    return pl.pallas_call(
        _tp_body,
        grid=(NTP,),
        in_specs=[pl.BlockSpec((EMB, W), lambda i: (0, i))],
        out_specs=pl.BlockSpec((W, 128), lambda i: (i, 0)),
        out_shape=jax.ShapeDtypeStruct((VOCAB, 128), jnp.float32),
    )(wtT)


def kernel(sentence, word_table, pos_table):
    wt128 = _tc_relayout(jnp.transpose(word_table, (1, 0)))
    idx = jnp.transpose(sentence, (1, 0)).reshape(ROWS // G, G)
    pos = lax.slice_in_dim(pos_table, 1, SEQ + 1, axis=0)
    mesh = plsc.VectorSubcoreMesh(core_axis_name="c", subcore_axis_name="s")
    out = pl.kernel(
        _emb_body,
        out_type=jax.ShapeDtypeStruct((ROWS, EMB), jnp.float32),
        mesh=mesh,
        compiler_params=pltpu.CompilerParams(use_tc_tiling_on_sc=False),
        scratch_types=[
            pltpu.VMEM((8, G), jnp.int32),
            pltpu.VMEM((8, G), jnp.int32),
            pltpu.VMEM((CHUNK, 128), jnp.float32),
            pltpu.VMEM((CHUNK, 128), jnp.float32),
            pltpu.VMEM((CHUNK, EMB), jnp.float32),
            pltpu.VMEM((CHUNK, EMB), jnp.float32),
            pltpu.VMEM((SEQ, EMB), jnp.float32),
            pltpu.SemaphoreType.DMA,
            pltpu.SemaphoreType.DMA,
            pltpu.SemaphoreType.DMA,
            pltpu.SemaphoreType.DMA,
        ],
    )(idx, wt128, pos)
    return out.reshape(BATCH, SEQ, EMB)


# speedup vs baseline: 4.4108x; 1.2297x over previous
import jax
import jax.numpy as jnp
from jax import lax
from jax.experimental import pallas as pl
from jax.experimental.pallas import tpu as pltpu
from jax.experimental.pallas import tpu_sc as plsc

VOCAB = 1000000
EMB = 64
SEQ = 200
BATCH = 1024

NC = 2
NS = 16
L = 16
NW = NC * NS
ROWS = SEQ * BATCH
RPW = ROWS // NW
CHUNK = SEQ
NCHUNK = RPW // CHUNK
G = 100
NG = CHUNK // G


def _emb_body(idx_hbm, table_hbm, pos_hbm, out_hbm,
              iv0, iv1, r0v, r1v, o0v, o1v, pos_v, g0, g1, w0, w1):
    wid = lax.axis_index("s") * NC + lax.axis_index("c")
    base = wid * (RPW // SEQ)
    ivs = [iv0, iv1]
    rows = [r0v, r1v]
    outs = [o0v, o1v]
    gsems = [g0, g1]
    osems = [w0, w1]
    pltpu.sync_copy(pos_hbm, pos_v)

    def stage_idx(grp):
        r0 = pl.multiple_of(wid * (RPW // G) + grp * 8, 8)
        pltpu.sync_copy(idx_hbm.at[pl.ds(r0, 8)], ivs[grp % 2])

    def gathers(c, p):
        return [
            pltpu.make_async_copy(
                table_hbm.at[ivs[(c // 4) % 2].at[2 * (c % 4) + k]],
                rows[p].at[pl.ds(G * k, G)],
                gsems[p],
            )
            for k in range(NG)
        ]

    def out_copy(c, p):
        return pltpu.make_async_copy(outs[p], out_hbm.at[pl.ds(base + c, 1)], osems[p])

    stage_idx(0)
    for cp in gathers(0, 0):
        cp.start()
    for c in range(NCHUNK):
        p = c % 2
        if c + 1 < NCHUNK:
            if (c + 1) % 4 == 0:
                stage_idx((c + 1) // 4)
            for cp in gathers(c + 1, (c + 1) % 2):
                cp.start()
        for cp in gathers(c, p):
            cp.wait()
        if c >= 2:
            out_copy(c - 2, p).wait()

        def body(t, carry):
            for j in range(EMB // L):
                outs[p][0, t, pl.ds(j * L, L)] = (
                    rows[p][t, pl.ds(j * L, L)] + pos_v[t, pl.ds(j * L, L)]
                )
            return carry

        lax.fori_loop(0, SEQ, body, 0)
        out_copy(c, p).start()
    out_copy(NCHUNK - 2, 0).wait()
    out_copy(NCHUNK - 1, 1).wait()


W = 8192
NTP = (VOCAB + W - 1) // W


def _tp_body(x_ref, o_ref):
    o_ref[:, 0:64] = x_ref[...].T


def _tc_relayout(wtT):
    return pl.pallas_call(
        _tp_body,
        grid=(NTP,),
        in_specs=[pl.BlockSpec((EMB, W), lambda i: (0, i))],
        out_specs=pl.BlockSpec((W, 128), lambda i: (i, 0)),
        out_shape=jax.ShapeDtypeStruct((VOCAB, 128), jnp.float32),
    )(wtT)


def kernel(sentence, word_table, pos_table):
    wt128 = _tc_relayout(jnp.transpose(word_table, (1, 0)))
    idx = jnp.transpose(sentence, (1, 0)).reshape(ROWS // G, G)
    pos = jnp.pad(
        lax.slice_in_dim(pos_table, 1, SEQ + 1, axis=0), ((0, 0), (0, 128 - EMB))
    )
    mesh = plsc.VectorSubcoreMesh(core_axis_name="c", subcore_axis_name="s")
    out = pl.kernel(
        _emb_body,
        out_type=jax.ShapeDtypeStruct((BATCH, SEQ, EMB), jnp.float32),
        mesh=mesh,
        compiler_params=pltpu.CompilerParams(
            use_tc_tiling_on_sc=True, needs_layout_passes=False
        ),
        scratch_types=[
            pltpu.VMEM((8, G), jnp.int32),
            pltpu.VMEM((8, G), jnp.int32),
            pltpu.VMEM((CHUNK, 128), jnp.float32),
            pltpu.VMEM((CHUNK, 128), jnp.float32),
            pltpu.VMEM((1, SEQ, EMB), jnp.float32),
            pltpu.VMEM((1, SEQ, EMB), jnp.float32),
            pltpu.VMEM((SEQ, 128), jnp.float32),
            pltpu.SemaphoreType.DMA,
            pltpu.SemaphoreType.DMA,
            pltpu.SemaphoreType.DMA,
            pltpu.SemaphoreType.DMA,
        ],
    )(idx, wt128, pos)
    return out


# speedup vs baseline: 4.6154x; 1.0464x over previous
import jax
import jax.numpy as jnp
from jax import lax
from jax.experimental import pallas as pl
from jax.experimental.pallas import tpu as pltpu
from jax.experimental.pallas import tpu_sc as plsc

VOCAB = 1000000
EMB = 64
SEQ = 200
BATCH = 1024

NC = 2
NS = 16
L = 16
NW = NC * NS
ROWS = SEQ * BATCH
RPW = ROWS // NW
CHUNK = SEQ
NCHUNK = RPW // CHUNK
G = 100
NG = CHUNK // G


def _emb_body(idx_hbm, table_hbm, pos_hbm, out_hbm,
              iv0, iv1, r0v, r1v, o0v, o1v, pos_v, g0, g1, w0, w1):
    wid = lax.axis_index("s") * NC + lax.axis_index("c")
    base = wid * (RPW // SEQ)
    ivs = [iv0, iv1]
    rows = [r0v, r1v]
    outs = [o0v, o1v]
    gsems = [g0, g1]
    osems = [w0, w1]
    pltpu.sync_copy(pos_hbm, pos_v)

    def stage_idx(grp):
        r0 = pl.multiple_of(wid * (RPW // G) + grp * 8, 8)
        pltpu.sync_copy(idx_hbm.at[pl.ds(r0, 8)], ivs[grp % 2])

    def gathers(c, p):
        return [
            pltpu.make_async_copy(
                table_hbm.at[ivs[(c // 4) % 2].at[2 * (c % 4) + k]],
                rows[p].at[pl.ds(G * k, G)],
                gsems[p],
            )
            for k in range(NG)
        ]

    def out_copy(c, p):
        return pltpu.make_async_copy(outs[p], out_hbm.at[pl.ds(base + c, 1)], osems[p])

    stage_idx(0)
    for cp in gathers(0, 0):
        cp.start()
    for c in range(NCHUNK):
        p = c % 2
        if c + 1 < NCHUNK:
            if (c + 1) % 4 == 0:
                stage_idx((c + 1) // 4)
            for cp in gathers(c + 1, (c + 1) % 2):
                cp.start()
        for cp in gathers(c, p):
            cp.wait()
        if c >= 2:
            out_copy(c - 2, p).wait()

        def body(t, carry):
            for j in range(EMB // L):
                outs[p][0, t, pl.ds(j * L, L)] = (
                    rows[p][t, pl.ds(j * L, L)] + pos_v[t, pl.ds(j * L, L)]
                )
            return carry

        lax.fori_loop(0, SEQ, body, 0)
        out_copy(c, p).start()
    out_copy(NCHUNK - 2, 0).wait()
    out_copy(NCHUNK - 1, 1).wait()


W = 16384
NTP = (VOCAB + W - 1) // W


def _tp_body(x_ref, o_ref):
    o_ref[:, 0:64] = x_ref[...].T


def _tc_relayout(wtT):
    return pl.pallas_call(
        _tp_body,
        grid=(NTP,),
        in_specs=[pl.BlockSpec((EMB, W), lambda i: (0, i))],
        out_specs=pl.BlockSpec((W, 128), lambda i: (i, 0)),
        out_shape=jax.ShapeDtypeStruct((VOCAB, 128), jnp.float32),
    )(wtT)


def kernel(sentence, word_table, pos_table):
    wt128 = _tc_relayout(jnp.transpose(word_table, (1, 0)))
    idx = jnp.transpose(sentence, (1, 0)).reshape(ROWS // G, G)
    pos = jnp.pad(
        lax.slice_in_dim(pos_table, 1, SEQ + 1, axis=0), ((0, 0), (0, 128 - EMB))
    )
    mesh = plsc.VectorSubcoreMesh(core_axis_name="c", subcore_axis_name="s")
    out = pl.kernel(
        _emb_body,
        out_type=jax.ShapeDtypeStruct((BATCH, SEQ, EMB), jnp.float32),
        mesh=mesh,
        compiler_params=pltpu.CompilerParams(
            use_tc_tiling_on_sc=True, needs_layout_passes=False
        ),
        scratch_types=[
            pltpu.VMEM((8, G), jnp.int32),
            pltpu.VMEM((8, G), jnp.int32),
            pltpu.VMEM((CHUNK, 128), jnp.float32),
            pltpu.VMEM((CHUNK, 128), jnp.float32),
            pltpu.VMEM((1, SEQ, EMB), jnp.float32),
            pltpu.VMEM((1, SEQ, EMB), jnp.float32),
            pltpu.VMEM((SEQ, 128), jnp.float32),
            pltpu.SemaphoreType.DMA,
            pltpu.SemaphoreType.DMA,
            pltpu.SemaphoreType.DMA,
            pltpu.SemaphoreType.DMA,
        ],
    )(idx, wt128, pos)
    return out


# speedup vs baseline: 4.6844x; 1.0150x over previous
import jax
import jax.numpy as jnp
from jax import lax
from jax.experimental import pallas as pl
from jax.experimental.pallas import tpu as pltpu
from jax.experimental.pallas import tpu_sc as plsc

VOCAB = 1000000
EMB = 64
SEQ = 200
BATCH = 1024

NC = 2
NS = 16
L = 16
NW = NC * NS
ROWS = SEQ * BATCH
RPW = ROWS // NW
CHUNK = SEQ
NCHUNK = RPW // CHUNK
G = 100
NG = CHUNK // G


def _emb_body(idx_hbm, table_hbm, pos_hbm, out_hbm,
              iv0, iv1, r0v, r1v, o0v, o1v, pos_v, g0, g1, w0, w1):
    wid = lax.axis_index("s") * NC + lax.axis_index("c")
    base = wid * (RPW // SEQ)
    ivs = [iv0, iv1]
    rows = [r0v, r1v]
    outs = [o0v, o1v]
    gsems = [g0, g1]
    osems = [w0, w1]
    pltpu.sync_copy(pos_hbm, pos_v)

    def stage_idx(grp):
        r0 = pl.multiple_of(wid * (RPW // G) + grp * 8, 8)
        pltpu.sync_copy(idx_hbm.at[pl.ds(r0, 8)], ivs[grp % 2])

    def gathers(c, p):
        return [
            pltpu.make_async_copy(
                table_hbm.at[ivs[(c // 4) % 2].at[2 * (c % 4) + k]],
                rows[p].at[pl.ds(G * k, G)],
                gsems[p],
            )
            for k in range(NG)
        ]

    def out_copy(c, p):
        return pltpu.make_async_copy(outs[p], out_hbm.at[pl.ds(base + c, 1)], osems[p])

    stage_idx(0)
    for cp in gathers(0, 0):
        cp.start()
    for c in range(NCHUNK):
        p = c % 2
        if c + 1 < NCHUNK:
            if (c + 1) % 4 == 0:
                stage_idx((c + 1) // 4)
            for cp in gathers(c + 1, (c + 1) % 2):
                cp.start()
        for cp in gathers(c, p):
            cp.wait()
        if c >= 2:
            out_copy(c - 2, p).wait()

        def body(t, carry):
            for j in range(EMB // L):
                outs[p][0, t, pl.ds(j * L, L)] = (
                    rows[p][t, pl.ds(j * L, L)] + pos_v[t, pl.ds(j * L, L)]
                )
            return carry

        lax.fori_loop(0, SEQ, body, 0)
        out_copy(c, p).start()
    out_copy(NCHUNK - 2, 0).wait()
    out_copy(NCHUNK - 1, 1).wait()


W = 32768
NTP = (VOCAB + W - 1) // W


def _tp_body(x_ref, o_ref):
    o_ref[:, 0:64] = x_ref[...].T


def _tc_relayout(wtT):
    return pl.pallas_call(
        _tp_body,
        grid=(NTP,),
        in_specs=[pl.BlockSpec((EMB, W), lambda i: (0, i))],
        out_specs=pl.BlockSpec((W, 128), lambda i: (i, 0)),
        out_shape=jax.ShapeDtypeStruct((VOCAB, 128), jnp.float32),
    )(wtT)


def kernel(sentence, word_table, pos_table):
    wt128 = _tc_relayout(jnp.transpose(word_table, (1, 0)))
    idx = jnp.transpose(sentence, (1, 0)).reshape(ROWS // G, G)
    pos = jnp.pad(
        lax.slice_in_dim(pos_table, 1, SEQ + 1, axis=0), ((0, 0), (0, 128 - EMB))
    )
    mesh = plsc.VectorSubcoreMesh(core_axis_name="c", subcore_axis_name="s")
    out = pl.kernel(
        _emb_body,
        out_type=jax.ShapeDtypeStruct((BATCH, SEQ, EMB), jnp.float32),
        mesh=mesh,
        compiler_params=pltpu.CompilerParams(
            use_tc_tiling_on_sc=True, needs_layout_passes=False
        ),
        scratch_types=[
            pltpu.VMEM((8, G), jnp.int32),
            pltpu.VMEM((8, G), jnp.int32),
            pltpu.VMEM((CHUNK, 128), jnp.float32),
            pltpu.VMEM((CHUNK, 128), jnp.float32),
            pltpu.VMEM((1, SEQ, EMB), jnp.float32),
            pltpu.VMEM((1, SEQ, EMB), jnp.float32),
            pltpu.VMEM((SEQ, 128), jnp.float32),
            pltpu.SemaphoreType.DMA,
            pltpu.SemaphoreType.DMA,
            pltpu.SemaphoreType.DMA,
            pltpu.SemaphoreType.DMA,
        ],
    )(idx, wt128, pos)
    return out
